# baseline (speedup 1.0000x reference)
import functools

import jax
import jax.numpy as jnp
from jax import lax
from jax.experimental import pallas as pl
from jax.experimental.pallas import tpu as pltpu
from jax.experimental.pallas import tpu_sc as plsc

_BATCH_SIZE = 64
_ROW_BLK = 4000
_N_TILES = 16


_OUT_COLS = 500


def _rowsum_body(a_ref, b_ref, o_ref):
    d = a_ref[...] - b_ref[...]
    rs = jnp.sum(d * d, axis=1)
    o_ref[...] = rs.reshape(o_ref.shape)


def _rowsums_tc(pred_v, adv):
    n, d = pred_v.shape
    blk = _ROW_BLK
    assert n % blk == 0 and blk % _OUT_COLS == 0
    rows = blk // _OUT_COLS
    assert rows % 8 == 0
    return pl.pallas_call(
        _rowsum_body,
        grid=(n // blk,),
        in_specs=[
            pl.BlockSpec((blk, d), lambda i: (i, 0)),
            pl.BlockSpec((blk, d), lambda i: (i, 0)),
        ],
        out_specs=pl.BlockSpec((rows, _OUT_COLS), lambda i: (i, 0)),
        out_shape=jax.ShapeDtypeStruct((n // _OUT_COLS, _OUT_COLS),
                                       jnp.float32),
    )(pred_v, adv)


def _chunking(n):
    chunk = ((n + _N_TILES * 16 - 1) // (_N_TILES * 16)) * 16
    assert chunk % 8 == 0 and (n - chunk) % 8 == 0
    return chunk, chunk // 16


def _sc_mesh():
    return plsc.VectorSubcoreMesh(
        core_axis_name="c", subcore_axis_name="s",
        num_cores=2, num_subcores=16)


def _counts_sc(batch, d_cols):
    n = batch.shape[0]
    chunk, steps = _chunking(n)
    nb = _BATCH_SIZE

    @functools.partial(
        pl.kernel,
        out_type=jax.ShapeDtypeStruct((nb,), jnp.float32),
        mesh=_sc_mesh(),
        compiler_params=pltpu.CompilerParams(needs_layout_passes=False),
        scratch_types=[
            pltpu.VMEM((chunk,), jnp.int32),
            pltpu.VMEM((16, nb), jnp.float32),
            pltpu.VMEM((2 * nb,), jnp.float32),
            pltpu.VMEM((_N_TILES, 2 * nb), jnp.float32),
            pltpu.VMEM((nb,), jnp.float32),
            pltpu.VMEM_SHARED((_N_TILES, 2 * nb), jnp.float32),
        ],
    )
    def cnt_kernel(batch_hbm, out_hbm, batch_v, accc, hist_v, tot_v, out_v,
                   shared):
        cid = lax.axis_index("c")
        sid = lax.axis_index("s")
        lanes = lax.iota(jnp.int32, 16)
        zeros16 = jnp.zeros((16,), jnp.float32)

        @pl.when(cid == 0)
        def _():
            own_lo = sid * chunk
            base = jnp.minimum(own_lo, n - chunk)
            pltpu.sync_copy(batch_hbm.at[pl.ds(base, chunk)], batch_v)
            for r in range(16):
                for k in range(nb // 16):
                    accc[r, pl.ds(k * 16, 16)] = zeros16
            ones = jnp.ones((16,), jnp.float32)

            def step(i, carry):
                b = batch_v[pl.ds(i * 16, 16)]
                gidx = base + i * 16 + lanes
                valid = gidx >= own_lo
                plsc.addupdate_scatter(accc, [lanes, b], ones, mask=valid)
                return carry

            lax.fori_loop(0, steps, step, 0, unroll=4)

            for k in range(nb // 16):
                c = accc[0, pl.ds(k * 16, 16)]
                for r in range(1, 16):
                    c = c + accc[r, pl.ds(k * 16, 16)]
                hist_v[pl.ds(k * 16, 16)] = c
            pltpu.sync_copy(hist_v, shared.at[sid])

        plsc.subcore_barrier()

        @pl.when(jnp.logical_and(cid == 0, sid == 0))
        def _():
            pltpu.sync_copy(shared, tot_v)
            dvec = jnp.full((16,), float(d_cols), jnp.float32)
            for k in range(nb // 16):
                c = tot_v[0, pl.ds(k * 16, 16)]
                for t in range(1, _N_TILES):
                    c = c + tot_v[t, pl.ds(k * 16, 16)]
                out_v[pl.ds(k * 16, 16)] = 1.0 / (dvec * jnp.maximum(c, 1.0))
            pltpu.sync_copy(out_v, out_hbm)

    return cnt_kernel(batch)


def _sums_sc(vals, batch, inv_denom):
    n = vals.shape[0]
    chunk, steps = _chunking(n)
    nb = _BATCH_SIZE

    @functools.partial(
        pl.kernel,
        out_type=jax.ShapeDtypeStruct((nb,), jnp.float32),
        mesh=_sc_mesh(),
        compiler_params=pltpu.CompilerParams(needs_layout_passes=False),
        scratch_types=[
            pltpu.VMEM((chunk,), jnp.float32),
            pltpu.VMEM((chunk,), jnp.int32),
            pltpu.VMEM((64, nb), jnp.float32),
            pltpu.VMEM((2 * nb,), jnp.float32),
            pltpu.VMEM((_N_TILES, 2 * nb), jnp.float32),
            pltpu.VMEM((nb,), jnp.float32),
            pltpu.VMEM((nb,), jnp.float32),
            pltpu.VMEM_SHARED((_N_TILES, 2 * nb), jnp.float32),
        ],
    )
    def sum_kernel(vals_hbm, batch_hbm, inv_hbm, out_hbm,
                   vals_v, batch_v, accs, hist_v, tot_v, inv_v, out_v,
                   shared):
        cid = lax.axis_index("c")
        sid = lax.axis_index("s")
        lanes = lax.iota(jnp.int32, 16)
        zeros16 = jnp.zeros((16,), jnp.float32)

        @pl.when(cid == 0)
        def _():
            own_lo = sid * chunk
            base = jnp.minimum(own_lo, n - chunk)
            pltpu.sync_copy(vals_hbm.at[pl.ds(base, chunk)], vals_v)
            pltpu.sync_copy(batch_hbm.at[pl.ds(base, chunk)], batch_v)
            for r in range(64):
                for k in range(nb // 16):
                    accs[r, pl.ds(k * 16, 16)] = zeros16

            def step(i, carry):
                v = vals_v[pl.ds(i * 16, 16)]
                b = batch_v[pl.ds(i * 16, 16)]
                gidx = base + i * 16 + lanes
                valid = gidx >= own_lo
                row = lanes + (i % 4) * 16
                plsc.addupdate_scatter(accs, [row, b], v, mask=valid)
                return carry

            lax.fori_loop(0, steps, step, 0, unroll=4)

            for k in range(nb // 16):
                s = accs[0, pl.ds(k * 16, 16)]
                for r in range(1, 64):
                    s = s + accs[r, pl.ds(k * 16, 16)]
                hist_v[pl.ds(k * 16, 16)] = s
            pltpu.sync_copy(hist_v, shared.at[sid])

        plsc.subcore_barrier()

        @pl.when(jnp.logical_and(cid == 0, sid == 0))
        def _():
            pltpu.sync_copy(shared, tot_v)
            pltpu.sync_copy(inv_hbm, inv_v)
            for k in range(nb // 16):
                s = tot_v[0, pl.ds(k * 16, 16)]
                for t in range(1, _N_TILES):
                    s = s + tot_v[t, pl.ds(k * 16, 16)]
                out_v[pl.ds(k * 16, 16)] = s * inv_v[pl.ds(k * 16, 16)]
            pltpu.sync_copy(out_v, out_hbm)

    return sum_kernel(vals, batch, inv_denom)


def kernel(pred_v, advection_field, batch):
    n, d = pred_v.shape
    bat = batch.astype(jnp.int32)
    inv_denom = _counts_sc(bat, d)
    rowsums = _rowsums_tc(pred_v, advection_field).reshape(n)
    return _sums_sc(rowsums, bat, inv_denom)


# speedup vs baseline: 1.0137x; 1.0137x over previous
import functools

import jax
import jax.numpy as jnp
from jax import lax
from jax.experimental import pallas as pl
from jax.experimental.pallas import tpu as pltpu
from jax.experimental.pallas import tpu_sc as plsc

_BATCH_SIZE = 64
_ROW_BLK = 4000
_N_TILES = 16


_OUT_COLS = 500


def _rowsum_body(a_ref, b_ref, o_ref):
    d = a_ref[...] - b_ref[...]
    rs = jnp.sum(d * d, axis=1)
    o_ref[...] = rs.reshape(o_ref.shape)


def _rowsums_tc(pred_v, adv):
    n, d = pred_v.shape
    blk = _ROW_BLK
    assert n % blk == 0 and blk % _OUT_COLS == 0
    rows = blk // _OUT_COLS
    assert rows % 8 == 0
    return pl.pallas_call(
        _rowsum_body,
        grid=(n // blk,),
        in_specs=[
            pl.BlockSpec((blk, d), lambda i: (i, 0)),
            pl.BlockSpec((blk, d), lambda i: (i, 0)),
        ],
        out_specs=pl.BlockSpec((rows, _OUT_COLS), lambda i: (i, 0)),
        out_shape=jax.ShapeDtypeStruct((n // _OUT_COLS, _OUT_COLS),
                                       jnp.float32),
    )(pred_v, adv)


def _chunking(n):
    chunk = ((n + _N_TILES * 16 - 1) // (_N_TILES * 16)) * 16
    assert chunk % 8 == 0 and (n - chunk) % 8 == 0
    return chunk, chunk // 16


def _sc_mesh():
    return plsc.VectorSubcoreMesh(
        core_axis_name="c", subcore_axis_name="s",
        num_cores=2, num_subcores=16)


def _counts_sc(batch, d_cols):
    n = batch.shape[0]
    chunk, steps = _chunking(n)
    nb = _BATCH_SIZE

    @functools.partial(
        pl.kernel,
        out_type=jax.ShapeDtypeStruct((nb,), jnp.float32),
        mesh=_sc_mesh(),
        compiler_params=pltpu.CompilerParams(needs_layout_passes=False),
        scratch_types=[
            pltpu.VMEM((chunk,), jnp.int32),
            pltpu.VMEM((16, nb), jnp.float32),
            pltpu.VMEM((2 * nb,), jnp.float32),
            pltpu.VMEM((_N_TILES, 2 * nb), jnp.float32),
            pltpu.VMEM((nb,), jnp.float32),
            pltpu.VMEM_SHARED((_N_TILES, 2 * nb), jnp.float32),
        ],
    )
    def cnt_kernel(batch_hbm, out_hbm, batch_v, accc, hist_v, tot_v, out_v,
                   shared):
        cid = lax.axis_index("c")
        sid = lax.axis_index("s")
        lanes = lax.iota(jnp.int32, 16)
        zeros16 = jnp.zeros((16,), jnp.float32)

        @pl.when(cid == 0)
        def _():
            own_lo = sid * chunk
            base = jnp.minimum(own_lo, n - chunk)
            pltpu.sync_copy(batch_hbm.at[pl.ds(base, chunk)], batch_v)
            for r in range(16):
                for k in range(nb // 16):
                    accc[r, pl.ds(k * 16, 16)] = zeros16
            ones = jnp.ones((16,), jnp.float32)

            def step(i, carry):
                b = batch_v[pl.ds(i * 16, 16)]
                gidx = base + i * 16 + lanes
                valid = gidx >= own_lo
                plsc.addupdate_scatter(accc, [lanes, b], ones, mask=valid)
                return carry

            lax.fori_loop(0, steps, step, 0, unroll=4)

            for k in range(nb // 16):
                c = accc[0, pl.ds(k * 16, 16)]
                for r in range(1, 16):
                    c = c + accc[r, pl.ds(k * 16, 16)]
                hist_v[pl.ds(k * 16, 16)] = c
            pltpu.sync_copy(hist_v, shared.at[sid])

        plsc.subcore_barrier()

        @pl.when(jnp.logical_and(cid == 0, sid == 0))
        def _():
            pltpu.sync_copy(shared, tot_v)
            dvec = jnp.full((16,), float(d_cols), jnp.float32)
            for k in range(nb // 16):
                c = tot_v[0, pl.ds(k * 16, 16)]
                for t in range(1, _N_TILES):
                    c = c + tot_v[t, pl.ds(k * 16, 16)]
                out_v[pl.ds(k * 16, 16)] = 1.0 / (dvec * jnp.maximum(c, 1.0))
            pltpu.sync_copy(out_v, out_hbm)

    return cnt_kernel(batch)


def _sums_sc(vals, batch, inv_denom):
    n = vals.shape[0]
    chunk, steps = _chunking(n)
    nb = _BATCH_SIZE

    @functools.partial(
        pl.kernel,
        out_type=jax.ShapeDtypeStruct((nb,), jnp.float32),
        mesh=_sc_mesh(),
        compiler_params=pltpu.CompilerParams(needs_layout_passes=False),
        scratch_types=[
            pltpu.VMEM((chunk,), jnp.float32),
            pltpu.VMEM((chunk,), jnp.int32),
            pltpu.VMEM((16, nb), jnp.float32),
            pltpu.VMEM((2 * nb,), jnp.float32),
            pltpu.VMEM((_N_TILES, 2 * nb), jnp.float32),
            pltpu.VMEM((nb,), jnp.float32),
            pltpu.VMEM((nb,), jnp.float32),
            pltpu.VMEM_SHARED((_N_TILES, 2 * nb), jnp.float32),
        ],
    )
    def sum_kernel(vals_hbm, batch_hbm, inv_hbm, out_hbm,
                   vals_v, batch_v, accs, hist_v, tot_v, inv_v, out_v,
                   shared):
        cid = lax.axis_index("c")
        sid = lax.axis_index("s")
        lanes = lax.iota(jnp.int32, 16)
        zeros16 = jnp.zeros((16,), jnp.float32)

        @pl.when(cid == 0)
        def _():
            own_lo = sid * chunk
            base = jnp.minimum(own_lo, n - chunk)
            pltpu.sync_copy(vals_hbm.at[pl.ds(base, chunk)], vals_v)
            pltpu.sync_copy(batch_hbm.at[pl.ds(base, chunk)], batch_v)
            for r in range(16):
                for k in range(nb // 16):
                    accs[r, pl.ds(k * 16, 16)] = zeros16

            def step(i, carry):
                v = vals_v[pl.ds(i * 16, 16)]
                b = batch_v[pl.ds(i * 16, 16)]
                gidx = base + i * 16 + lanes
                valid = gidx >= own_lo
                plsc.addupdate_scatter(accs, [lanes, b], v, mask=valid)
                return carry

            lax.fori_loop(0, steps, step, 0, unroll=4)

            for k in range(nb // 16):
                s = accs[0, pl.ds(k * 16, 16)]
                for r in range(1, 16):
                    s = s + accs[r, pl.ds(k * 16, 16)]
                hist_v[pl.ds(k * 16, 16)] = s
            pltpu.sync_copy(hist_v, shared.at[sid])

        plsc.subcore_barrier()

        @pl.when(jnp.logical_and(cid == 0, sid == 0))
        def _():
            pltpu.sync_copy(shared, tot_v)
            pltpu.sync_copy(inv_hbm, inv_v)
            for k in range(nb // 16):
                s = tot_v[0, pl.ds(k * 16, 16)]
                for t in range(1, _N_TILES):
                    s = s + tot_v[t, pl.ds(k * 16, 16)]
                out_v[pl.ds(k * 16, 16)] = s * inv_v[pl.ds(k * 16, 16)]
            pltpu.sync_copy(out_v, out_hbm)

    return sum_kernel(vals, batch, inv_denom)


def kernel(pred_v, advection_field, batch):
    n, d = pred_v.shape
    bat = batch.astype(jnp.int32)
    inv_denom = _counts_sc(bat, d)
    rowsums = _rowsums_tc(pred_v, advection_field).reshape(n)
    return _sums_sc(rowsums, bat, inv_denom)
